# baseline (speedup 1.0000x reference)
import functools

import jax
import jax.numpy as jnp
import numpy as np
from jax import lax
from jax.experimental import pallas as pl
from jax.experimental.pallas import tpu as pltpu
from jax.experimental.pallas import tpu_sc as plsc

EPS = float(np.finfo(np.float32).eps)
N = 10000
K = 16
E1 = 3
D = 128
NA = 100
HW = 304
NPADL = 10240

NC, NS, L = 2, 16, 16
NW = NC * NS
CH = 128
NCHUNK = NPADL // CH
NG_MAX = -(-NCHUNK // NW)


def _sc_hist_body(nf_hbm, idx_hbm, idxt_hbm, c_hbm, nf_v, idx_v0, idx_v1,
                  h_v0, h_v1, isem0, isem1, osem0, osem1, nfsem):
    wid = lax.axis_index("s") * NC + lax.axis_index("c")
    ng = jnp.where(wid < NCHUNK - (NG_MAX - 1) * NW, NG_MAX, NG_MAX - 1)
    iota = lax.iota(jnp.int32, L)
    ones = jnp.ones((L,), jnp.float32)
    zeros = jnp.zeros((L,), jnp.float32)
    idx_vs = (idx_v0, idx_v1)
    h_vs = (h_v0, h_v1)
    isems = (isem0, isem1)
    osems = (osem0, osem1)
    NFULL = N // CH

    def _idx_full(i, p):
        return pltpu.make_async_copy(
            idx_hbm.at[:, :, pl.ds((wid + NW * i) * CH, CH)], idx_vs[p],
            isems[p])

    def _idx_tail(i, p):
        cid = wid + NW * i
        return pltpu.make_async_copy(
            idxt_hbm.at[:, :, pl.ds((cid - NFULL) * CH, CH)], idx_vs[p],
            isems[p])

    def idx_issue(i, p, op):
        cid = wid + NW * i

        @pl.when(cid < NFULL)
        def _full():
            op(_idx_full(i, p))

        @pl.when(cid >= NFULL)
        def _tail():
            op(_idx_tail(i, p))

    def out_copy(i, p):
        return pltpu.make_async_copy(
            h_vs[p], c_hbm.at[:, pl.ds((wid + NW * i) * CH, CH)], osems[p])

    idx_issue(0, 0, lambda d: d.start())
    nf_dma = pltpu.make_async_copy(nf_hbm.at[0], nf_v, nfsem)
    nf_dma.start()
    ZR = 16

    for i in range(NG_MAX):
        p = i % 2

        @pl.when(i < ng)
        def _chunk(i=i, p=p):
            @pl.when(i + 1 < ng)
            def _prefetch():
                idx_issue(i + 1, 1 - p, lambda d: d.start())

            @pl.when(i >= 2)
            def _drain():
                out_copy(i - 2, p).wait()

            @plsc.parallel_loop(0, HW // ZR)
            def _zrow(r):
                for rr in range(ZR):
                    for cc in range(CH // L):
                        h_vs[p][r * ZR + rr, pl.ds(cc * L, L)] = zeros

            if i == 0:
                nf_dma.wait()
            idx_issue(i, p, lambda d: d.wait())

            @plsc.parallel_loop(0, CH // L, unroll=4)
            def _scatter(sub):
                cols = sub * L + iota
                for e in range(E1):
                    for k in range(K):
                        nbr = idx_vs[p][e, k, pl.ds(sub * L, L)]
                        cls = plsc.load_gather(nf_v, [nbr])
                        plsc.addupdate_scatter(
                            h_vs[p], [cls + (NA * e), cols], ones)

            out_copy(i, p).start()

    for p in range(2):
        out_copy(0, p).wait()


@functools.cache
def _sc_hist():
    return pl.kernel(
        _sc_hist_body,
        out_type=jax.ShapeDtypeStruct((HW, NPADL), jnp.float32),
        mesh=plsc.VectorSubcoreMesh(core_axis_name="c", subcore_axis_name="s",
                                    num_cores=NC, num_subcores=NS),
        compiler_params=pltpu.CompilerParams(needs_layout_passes=False),
        scratch_types=[
            pltpu.VMEM((N,), jnp.int32),
            pltpu.VMEM((E1, K, CH), jnp.int32),
            pltpu.VMEM((E1, K, CH), jnp.int32),
            pltpu.VMEM((HW, CH), jnp.float32),
            pltpu.VMEM((HW, CH), jnp.float32),
            pltpu.SemaphoreType.DMA,
            pltpu.SemaphoreType.DMA,
            pltpu.SemaphoreType.DMA,
            pltpu.SemaphoreType.DMA,
            pltpu.SemaphoreType.DMA,
        ],
    )


BLKN = 1280


def _tc_body(c_ref, mask_ref, embt_ref, w0t_ref, b0_ref, woutt_ref, bout_ref,
             watt_ref, batt_ref, out_ref, acc_ref, mt_ref):
    b = pl.program_id(0)

    @pl.when(b == 0)
    def _init():
        mt_ref[:] = jnp.zeros((D, HW), jnp.float32)
        for e in range(E1):
            me = jnp.dot(w0t_ref[e], embt_ref[:],
                         preferred_element_type=jnp.float32)
            mt_ref[:, NA * e:NA * (e + 1)] = me[:, :NA]
        acc_ref[:] = jnp.zeros_like(acc_ref)

    mask = mask_ref[:]
    hp = jnp.dot(mt_ref[:], c_ref[:], preferred_element_type=jnp.float32)
    hp = hp * (mask * (1.0 / K)) + b0_ref[:]
    h = jnp.maximum(hp, 0.0)
    nrm = jnp.sqrt(jnp.sum(h * h, axis=0, keepdims=True))
    h = h / (nrm + EPS)
    y = jnp.dot(woutt_ref[:], h, preferred_element_type=jnp.float32) + bout_ref[:]
    att = jax.nn.sigmoid(
        jnp.sum(h * watt_ref[:], axis=0, keepdims=True) + batt_ref[0, 0])
    part = lax.dot_general(att * mask, y, (((1,), (1,)), ((), ())),
                           preferred_element_type=jnp.float32)
    acc_ref[:] = acc_ref[:] + part

    @pl.when(b == pl.num_programs(0) - 1)
    def _fin():
        out_ref[:] = acc_ref[:] * (1.0 / N)


_tc_head = pl.pallas_call(
    _tc_body,
    grid=(NPADL // BLKN,),
    in_specs=[
        pl.BlockSpec((HW, BLKN), lambda b: (0, b)),
        pl.BlockSpec((1, BLKN), lambda b: (0, b)),
        pl.BlockSpec((D, D), lambda b: (0, 0)),
        pl.BlockSpec((E1, D, D), lambda b: (0, 0, 0)),
        pl.BlockSpec((D, 1), lambda b: (0, 0)),
        pl.BlockSpec((D, D), lambda b: (0, 0)),
        pl.BlockSpec((D, 1), lambda b: (0, 0)),
        pl.BlockSpec((D, 1), lambda b: (0, 0)),
        pl.BlockSpec((1, 1), lambda b: (0, 0)),
    ],
    out_specs=pl.BlockSpec((1, D), lambda b: (0, 0)),
    out_shape=jax.ShapeDtypeStruct((1, D), jnp.float32),
    scratch_shapes=[
        pltpu.VMEM((1, D), jnp.float32),
        pltpu.VMEM((D, HW), jnp.float32),
    ],
)


def kernel(node_feat, nn_idx, nonempty_mask, emb, W0, b0, Wout, bout, Watt, batt):
    idx_t = nn_idx.transpose(0, 3, 2, 1).reshape(E1, K, N)
    idx_tail = jnp.pad(idx_t[:, :, (N // CH) * CH:],
                       ((0, 0), (0, 0), (0, 2 * CH - N % CH)))

    counts_t = _sc_hist()(node_feat, idx_t, idx_tail)

    mask_t = jnp.pad(nonempty_mask.reshape(1, N), ((0, 0), (0, NPADL - N)))
    embt_p = jnp.zeros((D, D), jnp.float32).at[:, :emb.shape[0]].set(emb.T)
    w0t = W0.reshape(E1, D, D).transpose(0, 2, 1)
    return _tc_head(
        counts_t, mask_t, embt_p, w0t,
        b0.reshape(D, 1), Wout.T, bout.reshape(D, 1),
        Watt, batt.reshape(1, 1),
    )


# speedup vs baseline: 1.0323x; 1.0323x over previous
import functools

import jax
import jax.numpy as jnp
import numpy as np
from jax import lax
from jax.experimental import pallas as pl
from jax.experimental.pallas import tpu as pltpu
from jax.experimental.pallas import tpu_sc as plsc

EPS = float(np.finfo(np.float32).eps)
N = 10000
K = 16
E1 = 3
D = 128
NA = 100
HW = 304
NPADL = 10240

NC, NS, L = 2, 16, 16
NW = NC * NS
CH = 128
NCHUNK = NPADL // CH
NG_MAX = -(-NCHUNK // NW)


def _sc_hist_body(nf_hbm, idx_hbm, idxt_hbm, c_hbm, nf_v, idx_v0, idx_v1,
                  h_v0, h_v1, isem0, isem1, osem0, osem1, nfsem):
    wid = lax.axis_index("s") * NC + lax.axis_index("c")
    ng = jnp.where(wid < NCHUNK - (NG_MAX - 1) * NW, NG_MAX, NG_MAX - 1)
    iota = lax.iota(jnp.int32, L)
    ones = jnp.ones((L,), jnp.float32)
    zeros = jnp.zeros((L,), jnp.float32)
    idx_vs = (idx_v0, idx_v1)
    h_vs = (h_v0, h_v1)
    isems = (isem0, isem1)
    osems = (osem0, osem1)
    NFULL = N // CH

    def _idx_full(i, p):
        return pltpu.make_async_copy(
            idx_hbm.at[:, :, pl.ds((wid + NW * i) * CH, CH)], idx_vs[p],
            isems[p])

    def _idx_tail(i, p):
        cid = wid + NW * i
        return pltpu.make_async_copy(
            idxt_hbm.at[:, :, pl.ds((cid - NFULL) * CH, CH)], idx_vs[p],
            isems[p])

    def idx_issue(i, p, op):
        cid = wid + NW * i

        @pl.when(cid < NFULL)
        def _full():
            op(_idx_full(i, p))

        @pl.when(cid >= NFULL)
        def _tail():
            op(_idx_tail(i, p))

    def out_copy(i, p):
        return pltpu.make_async_copy(
            h_vs[p], c_hbm.at[:, pl.ds((wid + NW * i) * CH, CH)], osems[p])

    idx_issue(0, 0, lambda d: d.start())
    nf_dma = pltpu.make_async_copy(nf_hbm.at[0], nf_v, nfsem)
    nf_dma.start()
    ZR = 16

    for i in range(NG_MAX):
        p = i % 2

        @pl.when(i < ng)
        def _chunk(i=i, p=p):
            @pl.when(i + 1 < ng)
            def _prefetch():
                idx_issue(i + 1, 1 - p, lambda d: d.start())

            @pl.when(i >= 2)
            def _drain():
                out_copy(i - 2, p).wait()

            @plsc.parallel_loop(0, HW // ZR)
            def _zrow(r):
                for rr in range(ZR):
                    for cc in range(CH // L):
                        h_vs[p][r * ZR + rr, pl.ds(cc * L, L)] = zeros

            if i == 0:
                nf_dma.wait()
            idx_issue(i, p, lambda d: d.wait())

            @plsc.parallel_loop(0, CH // L, unroll=4)
            def _scatter(sub):
                cols = sub * L + iota
                for e in range(E1):
                    for k in range(K):
                        nbr = idx_vs[p][e, k, pl.ds(sub * L, L)]
                        cls = plsc.load_gather(nf_v, [nbr])
                        plsc.addupdate_scatter(
                            h_vs[p], [cls + (NA * e), cols], ones)

            out_copy(i, p).start()

    for p in range(2):
        out_copy(0, p).wait()


@functools.cache
def _sc_hist():
    return pl.kernel(
        _sc_hist_body,
        out_type=jax.ShapeDtypeStruct((HW, NPADL), jnp.float32),
        mesh=plsc.VectorSubcoreMesh(core_axis_name="c", subcore_axis_name="s",
                                    num_cores=NC, num_subcores=NS),
        compiler_params=pltpu.CompilerParams(needs_layout_passes=False),
        scratch_types=[
            pltpu.VMEM((N,), jnp.int32),
            pltpu.VMEM((E1, K, CH), jnp.int32),
            pltpu.VMEM((E1, K, CH), jnp.int32),
            pltpu.VMEM((HW, CH), jnp.float32),
            pltpu.VMEM((HW, CH), jnp.float32),
            pltpu.SemaphoreType.DMA,
            pltpu.SemaphoreType.DMA,
            pltpu.SemaphoreType.DMA,
            pltpu.SemaphoreType.DMA,
            pltpu.SemaphoreType.DMA,
        ],
    )


BLKN = 2048


def _tc_body(c_ref, mask_ref, embt_ref, w0t_ref, b0_ref, woutt_ref, bout_ref,
             watt_ref, batt_ref, out_ref, acc_ref, mt_ref):
    b = pl.program_id(0)

    @pl.when(b == 0)
    def _init():
        mt_ref[:] = jnp.zeros((D, HW), jnp.float32)
        for e in range(E1):
            me = jnp.dot(w0t_ref[e], embt_ref[:],
                         preferred_element_type=jnp.float32)
            mt_ref[:, NA * e:NA * (e + 1)] = me[:, :NA]
        acc_ref[:] = jnp.zeros_like(acc_ref)

    mask = mask_ref[:]
    hp = jnp.dot(mt_ref[:], c_ref[:], preferred_element_type=jnp.float32)
    hp = hp * (mask * (1.0 / K)) + b0_ref[:]
    h = jnp.maximum(hp, 0.0)
    nrm = jnp.sqrt(jnp.sum(h * h, axis=0, keepdims=True))
    h = h / (nrm + EPS)
    y = jnp.dot(woutt_ref[:], h, preferred_element_type=jnp.float32) + bout_ref[:]
    att = jax.nn.sigmoid(
        jnp.sum(h * watt_ref[:], axis=0, keepdims=True) + batt_ref[0, 0])
    part = lax.dot_general(att * mask, y, (((1,), (1,)), ((), ())),
                           preferred_element_type=jnp.float32)
    acc_ref[:] = acc_ref[:] + part

    @pl.when(b == pl.num_programs(0) - 1)
    def _fin():
        out_ref[:] = acc_ref[:] * (1.0 / N)


_tc_head = pl.pallas_call(
    _tc_body,
    grid=(NPADL // BLKN,),
    in_specs=[
        pl.BlockSpec((HW, BLKN), lambda b: (0, b)),
        pl.BlockSpec((1, BLKN), lambda b: (0, b)),
        pl.BlockSpec((D, D), lambda b: (0, 0)),
        pl.BlockSpec((E1, D, D), lambda b: (0, 0, 0)),
        pl.BlockSpec((D, 1), lambda b: (0, 0)),
        pl.BlockSpec((D, D), lambda b: (0, 0)),
        pl.BlockSpec((D, 1), lambda b: (0, 0)),
        pl.BlockSpec((D, 1), lambda b: (0, 0)),
        pl.BlockSpec((1, 1), lambda b: (0, 0)),
    ],
    out_specs=pl.BlockSpec((1, D), lambda b: (0, 0)),
    out_shape=jax.ShapeDtypeStruct((1, D), jnp.float32),
    scratch_shapes=[
        pltpu.VMEM((1, D), jnp.float32),
        pltpu.VMEM((D, HW), jnp.float32),
    ],
)


def kernel(node_feat, nn_idx, nonempty_mask, emb, W0, b0, Wout, bout, Watt, batt):
    idx_t = nn_idx.transpose(0, 3, 2, 1).reshape(E1, K, N)
    idx_tail = jnp.pad(idx_t[:, :, (N // CH) * CH:],
                       ((0, 0), (0, 0), (0, 2 * CH - N % CH)))

    counts_t = _sc_hist()(node_feat, idx_t, idx_tail)

    mask_t = jnp.pad(nonempty_mask.reshape(1, N), ((0, 0), (0, NPADL - N)))
    embt_p = jnp.zeros((D, D), jnp.float32).at[:, :emb.shape[0]].set(emb.T)
    w0t = W0.reshape(E1, D, D).transpose(0, 2, 1)
    return _tc_head(
        counts_t, mask_t, embt_p, w0t,
        b0.reshape(D, 1), Wout.T, bout.reshape(D, 1),
        Watt, batt.reshape(1, 1),
    )


# speedup vs baseline: 1.1610x; 1.1246x over previous
import functools

import jax
import jax.numpy as jnp
import numpy as np
from jax import lax
from jax.experimental import pallas as pl
from jax.experimental.pallas import tpu as pltpu
from jax.experimental.pallas import tpu_sc as plsc

EPS = float(np.finfo(np.float32).eps)
N = 10000
K = 16
E1 = 3
D = 128
NA = 100
HW = 304
NPADL = 10240

NC, NS, L = 2, 16, 16
NW = NC * NS
CH = 128
NCHUNK = NPADL // CH
NG_MAX = -(-NCHUNK // NW)


def _sc_hist_body(nf_hbm, idx_hbm, idxt_hbm, c_hbm, nf_v, idx_v, h_v,
                  isem, osem, nfsem):
    wid = lax.axis_index("s") * NC + lax.axis_index("c")
    ng = jnp.where(wid < NCHUNK - (NG_MAX - 1) * NW, NG_MAX, NG_MAX - 1)
    iota = lax.iota(jnp.int32, L)
    ones = jnp.ones((L,), jnp.float32)
    zeros = jnp.zeros((L,), jnp.float32)
    NFULL = N // CH

    def idx_issue(i, p, op):
        cid = wid + NW * i

        @pl.when(cid < NFULL)
        def _full():
            op(pltpu.make_async_copy(
                idx_hbm.at[:, :, pl.ds(cid * CH, CH)], idx_v.at[p],
                isem.at[p]))

        @pl.when(cid >= NFULL)
        def _tail():
            op(pltpu.make_async_copy(
                idxt_hbm.at[:, :, pl.ds((cid - NFULL) * CH, CH)], idx_v.at[p],
                isem.at[p]))

    def out_copy(i, p):
        return pltpu.make_async_copy(
            h_v.at[p], c_hbm.at[:, pl.ds((wid + NW * i) * CH, CH)], osem.at[p])

    idx_issue(0, 0, lambda d: d.start())
    nf_dma = pltpu.make_async_copy(nf_hbm.at[0], nf_v, nfsem)
    nf_dma.start()
    ZR = 16

    def chunk(i, carry):
        p = lax.rem(i, 2)

        @pl.when(i + 1 < ng)
        def _prefetch():
            idx_issue(i + 1, 1 - p, lambda d: d.start())

        @pl.when(i >= 2)
        def _drain():
            out_copy(i - 2, p).wait()

        @plsc.parallel_loop(0, HW // ZR)
        def _zrow(r):
            for rr in range(ZR):
                for cc in range(CH // L):
                    h_v[p, r * ZR + rr, pl.ds(cc * L, L)] = zeros

        @pl.when(i == 0)
        def _nf():
            nf_dma.wait()

        idx_issue(i, p, lambda d: d.wait())

        @plsc.parallel_loop(0, CH // L, unroll=4)
        def _scatter(sub):
            cols = sub * L + iota
            for e in range(E1):
                for k in range(K):
                    nbr = idx_v[p, e, k, pl.ds(sub * L, L)]
                    cls = plsc.load_gather(nf_v, [nbr])
                    plsc.addupdate_scatter(
                        h_v.at[p], [cls + (NA * e), cols], ones)

        out_copy(i, p).start()
        return carry

    lax.fori_loop(0, ng, chunk, 0)
    for p in range(2):
        out_copy(0, p).wait()


@functools.cache
def _sc_hist():
    return pl.kernel(
        _sc_hist_body,
        out_type=jax.ShapeDtypeStruct((HW, NPADL), jnp.float32),
        mesh=plsc.VectorSubcoreMesh(core_axis_name="c", subcore_axis_name="s",
                                    num_cores=NC, num_subcores=NS),
        compiler_params=pltpu.CompilerParams(needs_layout_passes=False),
        scratch_types=[
            pltpu.VMEM((N,), jnp.int32),
            pltpu.VMEM((2, E1, K, CH), jnp.int32),
            pltpu.VMEM((2, HW, CH), jnp.float32),
            pltpu.SemaphoreType.DMA((2,)),
            pltpu.SemaphoreType.DMA((2,)),
            pltpu.SemaphoreType.DMA,
        ],
    )


BLKN = 2048


def _tc_body(c_ref, mask_ref, embt_ref, w0t_ref, b0_ref, woutt_ref, bout_ref,
             watt_ref, batt_ref, out_ref, acc_ref, mt_ref):
    b = pl.program_id(0)

    @pl.when(b == 0)
    def _init():
        mt_ref[:] = jnp.zeros((D, HW), jnp.float32)
        for e in range(E1):
            me = jnp.dot(w0t_ref[e], embt_ref[:],
                         preferred_element_type=jnp.float32)
            mt_ref[:, NA * e:NA * (e + 1)] = me[:, :NA]
        acc_ref[:] = jnp.zeros_like(acc_ref)

    mask = mask_ref[:]
    hp = jnp.dot(mt_ref[:], c_ref[:], preferred_element_type=jnp.float32)
    hp = hp * (mask * (1.0 / K)) + b0_ref[:]
    h = jnp.maximum(hp, 0.0)
    nrm = jnp.sqrt(jnp.sum(h * h, axis=0, keepdims=True))
    h = h / (nrm + EPS)
    y = jnp.dot(woutt_ref[:], h, preferred_element_type=jnp.float32) + bout_ref[:]
    att = jax.nn.sigmoid(
        jnp.sum(h * watt_ref[:], axis=0, keepdims=True) + batt_ref[0, 0])
    part = lax.dot_general(att * mask, y, (((1,), (1,)), ((), ())),
                           preferred_element_type=jnp.float32)
    acc_ref[:] = acc_ref[:] + part

    @pl.when(b == pl.num_programs(0) - 1)
    def _fin():
        out_ref[:] = acc_ref[:] * (1.0 / N)


_tc_head = pl.pallas_call(
    _tc_body,
    grid=(NPADL // BLKN,),
    in_specs=[
        pl.BlockSpec((HW, BLKN), lambda b: (0, b)),
        pl.BlockSpec((1, BLKN), lambda b: (0, b)),
        pl.BlockSpec((D, D), lambda b: (0, 0)),
        pl.BlockSpec((E1, D, D), lambda b: (0, 0, 0)),
        pl.BlockSpec((D, 1), lambda b: (0, 0)),
        pl.BlockSpec((D, D), lambda b: (0, 0)),
        pl.BlockSpec((D, 1), lambda b: (0, 0)),
        pl.BlockSpec((D, 1), lambda b: (0, 0)),
        pl.BlockSpec((1, 1), lambda b: (0, 0)),
    ],
    out_specs=pl.BlockSpec((1, D), lambda b: (0, 0)),
    out_shape=jax.ShapeDtypeStruct((1, D), jnp.float32),
    scratch_shapes=[
        pltpu.VMEM((1, D), jnp.float32),
        pltpu.VMEM((D, HW), jnp.float32),
    ],
)


def kernel(node_feat, nn_idx, nonempty_mask, emb, W0, b0, Wout, bout, Watt, batt):
    idx_t = nn_idx.transpose(0, 3, 2, 1).reshape(E1, K, N)
    idx_tail = jnp.pad(idx_t[:, :, (N // CH) * CH:],
                       ((0, 0), (0, 0), (0, 2 * CH - N % CH)))

    counts_t = _sc_hist()(node_feat, idx_t, idx_tail)

    mask_t = jnp.pad(nonempty_mask.reshape(1, N), ((0, 0), (0, NPADL - N)))
    embt_p = jnp.zeros((D, D), jnp.float32).at[:, :emb.shape[0]].set(emb.T)
    w0t = W0.reshape(E1, D, D).transpose(0, 2, 1)
    return _tc_head(
        counts_t, mask_t, embt_p, w0t,
        b0.reshape(D, 1), Wout.T, bout.reshape(D, 1),
        Watt, batt.reshape(1, 1),
    )
